# baseline (device time: 22225 ns/iter reference)
import jax
import jax.numpy as jnp
from jax import lax
from jax.experimental import pallas as pl
from jax.experimental.pallas import tpu as pltpu

X_SIZE = 2


def kernel(x, router, W1, W2):
    t_per, d = x.shape
    e_per, _, f = W1.shape
    n_exp = e_per * X_SIZE
    half = t_per // 2

    def body(x_hbm, router_hbm, w1_hbm, w2_hbm, out_hbm,
             ro_ref, xb_ref, xo_ref, gm_ref, go_ref, pb_ref, po_ref,
             xv_ref, rv_ref, w1v_ref, w2v_ref, ob_ref,
             send_sems, recv_sems, copy_sems):
        my_x = lax.axis_index("x")
        my_y = lax.axis_index("y")
        my_z = lax.axis_index("z")
        partner = (1 - my_x, my_y, my_z)

        def rdma(src, dst, i):
            return pltpu.make_async_remote_copy(
                src_ref=src, dst_ref=dst,
                send_sem=send_sems.at[i], recv_sem=recv_sems.at[i],
                device_id=partner, device_id_type=pl.DeviceIdType.MESH)

        cp_x = pltpu.make_async_copy(x_hbm, xv_ref, copy_sems.at[0])
        cp_x.start()
        cp_r = pltpu.make_async_copy(router_hbm, rv_ref, copy_sems.at[1])
        cp_r.start()
        cp_w = []
        for j in range(e_per):
            c1 = pltpu.make_async_copy(w1_hbm.at[j], w1v_ref.at[j],
                                       copy_sems.at[2 + 2 * j])
            c1.start()
            c2 = pltpu.make_async_copy(w2_hbm.at[j], w2v_ref.at[j],
                                       copy_sems.at[3 + 2 * j])
            c2.start()
            cp_w.append((c1, c2))

        barrier_sem = pltpu.get_barrier_semaphore()
        pl.semaphore_signal(barrier_sem, inc=1, device_id=partner,
                            device_id_type=pl.DeviceIdType.MESH)

        cp_x.wait()
        x_mine = xv_ref[...]
        xb = x_mine.astype(jnp.bfloat16)
        xb_ref[...] = xb

        pl.semaphore_wait(barrier_sem, 1)

        cp_r.wait()
        r_router = rdma(rv_ref, ro_ref, 0)
        r_router.start()
        r_tok1 = rdma(xb_ref.at[pl.ds(0, half), :],
                      xo_ref.at[pl.ds(0, half), :], 1)
        r_tok1.start()

        r_router.wait()
        is0 = my_x == 0
        r_mine = rv_ref[...]
        r_oth = ro_ref[...]
        r_full = jnp.where(is0,
                           jnp.concatenate([r_mine, r_oth], axis=1),
                           jnp.concatenate([r_oth, r_mine], axis=1))
        gm = jnp.dot(x_mine, r_full, preferred_element_type=jnp.float32)
        gm_ref[...] = gm
        r_gate = rdma(gm_ref, go_ref, 2)
        r_gate.start()
        r_tok2 = rdma(xb_ref.at[pl.ds(half, half), :],
                      xo_ref.at[pl.ds(half, half), :], 3)
        r_tok2.start()

        def expert_weights(gates):
            i1 = jnp.argmax(gates, axis=1)
            m1 = jnp.max(gates, axis=1)
            cols = lax.broadcasted_iota(jnp.int32, gates.shape, 1)
            masked = jnp.where(cols == i1[:, None], -jnp.inf, gates)
            i2 = jnp.argmax(masked, axis=1)
            m2 = jnp.max(masked, axis=1)
            e2 = jnp.exp(m2 - m1)
            w_top = 1.0 / (1.0 + e2)
            w_snd = e2 / (1.0 + e2)
            ws = []
            for j in range(e_per):
                e_glob = my_x * e_per + j
                ws.append((jnp.where(i1 == e_glob, w_top, 0.0)
                           + jnp.where(i2 == e_glob, w_snd, 0.0))[:, None])
            return ws

        w_mine = expert_weights(gm)

        def y_block(tok_b, w1b, w2b, w_rows):
            h = jnp.maximum(
                jnp.dot(tok_b, w1b, preferred_element_type=jnp.float32),
                0.0).astype(jnp.bfloat16)
            return jnp.dot(h, w2b,
                           preferred_element_type=jnp.float32) * w_rows

        cp_w[0][0].wait()
        w1b0 = w1v_ref[0].astype(jnp.bfloat16)
        cp_w[0][1].wait()
        w2b0 = w2v_ref[0].astype(jnp.bfloat16)

        r_tok1.wait()
        r_gate.wait()
        w_for = expert_weights(go_ref[...])
        xo1 = xo_ref[pl.ds(0, half), :]
        f1_e0 = y_block(xo1, w1b0, w2b0, w_for[0][:half])
        r_tok2.wait()
        xo2 = xo_ref[pl.ds(half, half), :]
        f2_e0 = y_block(xo2, w1b0, w2b0, w_for[0][half:])
        my1_e0 = y_block(xb[:half], w1b0, w2b0, w_mine[0][:half])
        my2_e0 = y_block(xb[half:], w1b0, w2b0, w_mine[0][half:])

        cp_w[1][0].wait()
        w1b1 = w1v_ref[1].astype(jnp.bfloat16)
        cp_w[1][1].wait()
        w2b1 = w2v_ref[1].astype(jnp.bfloat16)

        f1_e1 = y_block(xo1, w1b1, w2b1, w_for[1][:half])
        pb_ref[pl.ds(0, half), :] = (f1_e0 + f1_e1).astype(jnp.bfloat16)
        r_par1 = rdma(pb_ref.at[pl.ds(0, half), :],
                      po_ref.at[pl.ds(0, half), :], 4)
        r_par1.start()

        f2_e1 = y_block(xo2, w1b1, w2b1, w_for[1][half:])
        pb_ref[pl.ds(half, half), :] = (f2_e0 + f2_e1).astype(jnp.bfloat16)
        r_par2 = rdma(pb_ref.at[pl.ds(half, half), :],
                      po_ref.at[pl.ds(half, half), :], 5)
        r_par2.start()

        my1_e1 = y_block(xb[:half], w1b1, w2b1, w_mine[1][:half])
        my2_e1 = y_block(xb[half:], w1b1, w2b1, w_mine[1][half:])
        acc_my = jnp.concatenate(
            [my1_e0 + my1_e1, my2_e0 + my2_e1], axis=0)

        r_par1.wait()
        r_par2.wait()
        ob_ref[...] = acc_my + po_ref[...].astype(jnp.float32)
        cp_out = pltpu.make_async_copy(ob_ref, out_hbm, copy_sems.at[0])
        cp_out.start()
        cp_out.wait()

    return pl.pallas_call(
        body,
        out_shape=jax.ShapeDtypeStruct((t_per, d), jnp.float32),
        in_specs=[pl.BlockSpec(memory_space=pltpu.MemorySpace.HBM)] * 4,
        out_specs=pl.BlockSpec(memory_space=pltpu.MemorySpace.HBM),
        scratch_shapes=[
            pltpu.VMEM((d, e_per), jnp.float32),
            pltpu.VMEM((t_per, d), jnp.bfloat16),
            pltpu.VMEM((t_per, d), jnp.bfloat16),
            pltpu.VMEM((t_per, n_exp), jnp.float32),
            pltpu.VMEM((t_per, n_exp), jnp.float32),
            pltpu.VMEM((t_per, d), jnp.bfloat16),
            pltpu.VMEM((t_per, d), jnp.bfloat16),
            pltpu.VMEM((t_per, d), jnp.float32),
            pltpu.VMEM((d, e_per), jnp.float32),
            pltpu.VMEM((e_per, d, f), jnp.float32),
            pltpu.VMEM((e_per, f, d), jnp.float32),
            pltpu.VMEM((t_per, d), jnp.float32),
            pltpu.SemaphoreType.DMA((6,)),
            pltpu.SemaphoreType.DMA((6,)),
            pltpu.SemaphoreType.DMA((6,)),
        ],
        compiler_params=pltpu.CompilerParams(collective_id=0),
    )(*(pltpu.with_memory_space_constraint(a, pltpu.MemorySpace.HBM)
        for a in (x, router, W1, W2)))


# device time: 19933 ns/iter; 1.1150x vs baseline; 1.1150x over previous
import jax
import jax.numpy as jnp
from jax import lax
from jax.experimental import pallas as pl
from jax.experimental.pallas import tpu as pltpu

X_SIZE = 2


def kernel(x, router, W1, W2):
    t_per, d = x.shape
    e_per, _, f = W1.shape
    n_exp = e_per * X_SIZE
    half = t_per // 2

    def body(x_hbm, router_hbm, w1_hbm, w2_hbm, out_hbm,
             ro_ref, xb_ref, xo_ref, gm_ref, go_ref, pb_ref, po_ref,
             xv_ref, rv_ref, w1v_ref, w2v_ref, w1c_ref, w2c_ref, ob_ref,
             send_sems, recv_sems, copy_sems):
        my_x = lax.axis_index("x")
        my_y = lax.axis_index("y")
        my_z = lax.axis_index("z")
        partner = (1 - my_x, my_y, my_z)

        def rdma(src, dst, i):
            return pltpu.make_async_remote_copy(
                src_ref=src, dst_ref=dst,
                send_sem=send_sems.at[i], recv_sem=recv_sems.at[i],
                device_id=partner, device_id_type=pl.DeviceIdType.MESH)

        cp_x = pltpu.make_async_copy(x_hbm, xv_ref, copy_sems.at[0])
        cp_x.start()
        cp_r = pltpu.make_async_copy(router_hbm, rv_ref, copy_sems.at[1])
        cp_r.start()
        cp_w = []
        for j in range(e_per):
            c1 = pltpu.make_async_copy(w1_hbm.at[j], w1v_ref.at[j],
                                       copy_sems.at[2 + 2 * j])
            c1.start()
            c2 = pltpu.make_async_copy(w2_hbm.at[j], w2v_ref.at[j],
                                       copy_sems.at[3 + 2 * j])
            c2.start()
            cp_w.append((c1, c2))

        barrier_sem = pltpu.get_barrier_semaphore()
        pl.semaphore_signal(barrier_sem, inc=1, device_id=partner,
                            device_id_type=pl.DeviceIdType.MESH)

        cp_x.wait()
        x_mine = xv_ref[...]
        xb = x_mine.astype(jnp.bfloat16)
        xb_ref[...] = xb

        pl.semaphore_wait(barrier_sem, 1)

        cp_r.wait()
        r_router = rdma(rv_ref, ro_ref, 0)
        r_router.start()
        r_tok1 = rdma(xb_ref.at[pl.ds(0, half), :],
                      xo_ref.at[pl.ds(0, half), :], 1)
        r_tok1.start()

        r_router.wait()
        is0 = my_x == 0
        r_mine = rv_ref[...]
        r_oth = ro_ref[...]
        r_full = jnp.where(is0,
                           jnp.concatenate([r_mine, r_oth], axis=1),
                           jnp.concatenate([r_oth, r_mine], axis=1))
        gm = jnp.dot(x_mine, r_full, preferred_element_type=jnp.float32)
        gm_ref[...] = gm
        r_gate = rdma(gm_ref, go_ref, 2)
        r_gate.start()
        r_tok2 = rdma(xb_ref.at[pl.ds(half, half), :],
                      xo_ref.at[pl.ds(half, half), :], 3)
        r_tok2.start()

        def expert_weights(gates):
            i1 = jnp.argmax(gates, axis=1)
            m1 = jnp.max(gates, axis=1)
            cols = lax.broadcasted_iota(jnp.int32, gates.shape, 1)
            masked = jnp.where(cols == i1[:, None], -jnp.inf, gates)
            i2 = jnp.argmax(masked, axis=1)
            m2 = jnp.max(masked, axis=1)
            e2 = jnp.exp(m2 - m1)
            w_top = 1.0 / (1.0 + e2)
            w_snd = e2 / (1.0 + e2)
            ws = []
            for j in range(e_per):
                e_glob = my_x * e_per + j
                w_e = (jnp.where(i1 == e_glob, w_top, 0.0)
                       + jnp.where(i2 == e_glob, w_snd, 0.0))
                ws.append(jnp.broadcast_to(
                    w_e[:, None].astype(jnp.bfloat16),
                    (gates.shape[0], f)))
            return jnp.concatenate(ws, axis=1)

        w_mine = expert_weights(gm)

        for j in range(e_per):
            cp_w[j][0].wait()
            w1c_ref[:, pl.ds(j * f, f)] = w1v_ref[j].astype(jnp.bfloat16)
            cp_w[j][1].wait()
            w2c_ref[pl.ds(j * f, f), :] = w2v_ref[j].astype(jnp.bfloat16)
        w1c = w1c_ref[...]
        w2c = w2c_ref[...]

        def block_out(tok_b, wcat):
            h = jnp.maximum(
                jnp.dot(tok_b, w1c, preferred_element_type=jnp.float32),
                0.0).astype(jnp.bfloat16)
            return jnp.dot(h * wcat, w2c, preferred_element_type=jnp.float32)

        acc_my1 = block_out(xb[:half], w_mine[:half])

        r_tok1.wait()
        r_gate.wait()
        w_for = expert_weights(go_ref[...])
        acc_f1 = block_out(xo_ref[pl.ds(0, half), :], w_for[:half])
        pb_ref[pl.ds(0, half), :] = acc_f1.astype(jnp.bfloat16)
        r_par1 = rdma(pb_ref.at[pl.ds(0, half), :],
                      po_ref.at[pl.ds(0, half), :], 4)
        r_par1.start()

        r_tok2.wait()
        acc_f2 = block_out(xo_ref[pl.ds(half, half), :], w_for[half:])
        pb_ref[pl.ds(half, half), :] = acc_f2.astype(jnp.bfloat16)
        r_par2 = rdma(pb_ref.at[pl.ds(half, half), :],
                      po_ref.at[pl.ds(half, half), :], 5)
        r_par2.start()

        acc_my2 = block_out(xb[half:], w_mine[half:])
        acc_my = jnp.concatenate([acc_my1, acc_my2], axis=0)

        r_par1.wait()
        r_par2.wait()
        ob_ref[...] = acc_my + po_ref[...].astype(jnp.float32)
        cp_out = pltpu.make_async_copy(ob_ref, out_hbm, copy_sems.at[0])
        cp_out.start()
        cp_out.wait()

    return pl.pallas_call(
        body,
        out_shape=jax.ShapeDtypeStruct((t_per, d), jnp.float32),
        in_specs=[pl.BlockSpec(memory_space=pltpu.MemorySpace.HBM)] * 4,
        out_specs=pl.BlockSpec(memory_space=pltpu.MemorySpace.HBM),
        scratch_shapes=[
            pltpu.VMEM((d, e_per), jnp.float32),
            pltpu.VMEM((t_per, d), jnp.bfloat16),
            pltpu.VMEM((t_per, d), jnp.bfloat16),
            pltpu.VMEM((t_per, n_exp), jnp.float32),
            pltpu.VMEM((t_per, n_exp), jnp.float32),
            pltpu.VMEM((t_per, d), jnp.bfloat16),
            pltpu.VMEM((t_per, d), jnp.bfloat16),
            pltpu.VMEM((t_per, d), jnp.float32),
            pltpu.VMEM((d, e_per), jnp.float32),
            pltpu.VMEM((e_per, d, f), jnp.float32),
            pltpu.VMEM((e_per, f, d), jnp.float32),
            pltpu.VMEM((d, e_per * f), jnp.bfloat16),
            pltpu.VMEM((e_per * f, d), jnp.bfloat16),
            pltpu.VMEM((t_per, d), jnp.float32),
            pltpu.SemaphoreType.DMA((6,)),
            pltpu.SemaphoreType.DMA((6,)),
            pltpu.SemaphoreType.DMA((6,)),
        ],
        compiler_params=pltpu.CompilerParams(collective_id=0),
    )(*(pltpu.with_memory_space_constraint(a, pltpu.MemorySpace.HBM)
        for a in (x, router, W1, W2)))


# device time: 17358 ns/iter; 1.2804x vs baseline; 1.1483x over previous
import jax
import jax.numpy as jnp
from jax import lax
from jax.experimental import pallas as pl
from jax.experimental.pallas import tpu as pltpu

X_SIZE = 2


def kernel(x, router, W1, W2):
    t_per, d = x.shape
    e_per, _, f = W1.shape
    n_exp = e_per * X_SIZE
    half = t_per // 2

    x_aug = jnp.concatenate([x, router.T], axis=0)

    def body(x_hbm, w1_hbm, w2_hbm, out_hbm,
             ro_ref, xb_ref, xo_ref, gm_ref, go_ref, pb_ref, po_ref,
             xv_ref, w1v_ref, w2v_ref, w1c_ref, w2c_ref, ob_ref,
             send_sems, recv_sems, copy_sems):
        my_x = lax.axis_index("x")
        my_y = lax.axis_index("y")
        my_z = lax.axis_index("z")
        partner = (1 - my_x, my_y, my_z)

        def rdma(src, dst, i):
            return pltpu.make_async_remote_copy(
                src_ref=src, dst_ref=dst,
                send_sem=send_sems.at[i], recv_sem=recv_sems.at[i],
                device_id=partner, device_id_type=pl.DeviceIdType.MESH)

        cp_x = pltpu.make_async_copy(x_hbm, xv_ref, copy_sems.at[0])
        cp_x.start()
        cp_w = []
        for j in range(e_per):
            c1 = pltpu.make_async_copy(w1_hbm.at[j], w1v_ref.at[j],
                                       copy_sems.at[2 + 2 * j])
            c1.start()
            c2 = pltpu.make_async_copy(w2_hbm.at[j], w2v_ref.at[j],
                                       copy_sems.at[3 + 2 * j])
            c2.start()
            cp_w.append((c1, c2))

        barrier_sem = pltpu.get_barrier_semaphore()
        pl.semaphore_signal(barrier_sem, inc=1, device_id=partner,
                            device_id_type=pl.DeviceIdType.MESH)

        cp_x.wait()
        x_mine = xv_ref[pl.ds(0, t_per), :]
        xb = x_mine.astype(jnp.bfloat16)
        xb_ref[...] = xb

        pl.semaphore_wait(barrier_sem, 1)

        r_router = rdma(xv_ref.at[pl.ds(t_per, e_per), :], ro_ref, 0)
        r_router.start()
        r_tok1 = rdma(xb_ref.at[pl.ds(0, half), :],
                      xo_ref.at[pl.ds(0, half), :], 1)
        r_tok1.start()

        r_router.wait()
        is0 = my_x == 0
        r_mine = xv_ref[pl.ds(t_per, e_per), :]
        r_oth = ro_ref[...]
        r_full = jnp.where(is0,
                           jnp.concatenate([r_mine, r_oth], axis=0),
                           jnp.concatenate([r_oth, r_mine], axis=0))
        gm = lax.dot_general(x_mine, r_full, (((1,), (1,)), ((), ())),
                             preferred_element_type=jnp.float32)
        gm_ref[...] = gm
        r_gate = rdma(gm_ref, go_ref, 2)
        r_gate.start()
        r_tok2 = rdma(xb_ref.at[pl.ds(half, half), :],
                      xo_ref.at[pl.ds(half, half), :], 3)
        r_tok2.start()

        def expert_weights(gates):
            i1 = jnp.argmax(gates, axis=1)
            m1 = jnp.max(gates, axis=1)
            cols = lax.broadcasted_iota(jnp.int32, gates.shape, 1)
            masked = jnp.where(cols == i1[:, None], -jnp.inf, gates)
            i2 = jnp.argmax(masked, axis=1)
            m2 = jnp.max(masked, axis=1)
            e2 = jnp.exp(m2 - m1)
            w_top = 1.0 / (1.0 + e2)
            w_snd = e2 / (1.0 + e2)
            ws = []
            for j in range(e_per):
                e_glob = my_x * e_per + j
                w_e = (jnp.where(i1 == e_glob, w_top, 0.0)
                       + jnp.where(i2 == e_glob, w_snd, 0.0))
                ws.append(jnp.broadcast_to(
                    w_e[:, None].astype(jnp.bfloat16),
                    (gates.shape[0], f)))
            return jnp.concatenate(ws, axis=1)

        w_mine = expert_weights(gm)

        for j in range(e_per):
            cp_w[j][0].wait()
            w1c_ref[:, pl.ds(j * f, f)] = w1v_ref[j].astype(jnp.bfloat16)
            cp_w[j][1].wait()
            w2c_ref[pl.ds(j * f, f), :] = w2v_ref[j].astype(jnp.bfloat16)
        w1c = w1c_ref[...]
        w2c = w2c_ref[...]

        def block_out(tok_b, wcat):
            h = jnp.maximum(
                jnp.dot(tok_b, w1c, preferred_element_type=jnp.float32),
                0.0).astype(jnp.bfloat16)
            return jnp.dot(h * wcat, w2c, preferred_element_type=jnp.float32)

        acc_my1 = block_out(xb[:half], w_mine[:half])

        r_tok1.wait()
        r_gate.wait()
        w_for = expert_weights(go_ref[...])
        acc_f1 = block_out(xo_ref[pl.ds(0, half), :], w_for[:half])
        pb_ref[pl.ds(0, half), :] = acc_f1.astype(jnp.bfloat16)
        r_par1 = rdma(pb_ref.at[pl.ds(0, half), :],
                      po_ref.at[pl.ds(0, half), :], 4)
        r_par1.start()

        r_tok2.wait()
        acc_f2 = block_out(xo_ref[pl.ds(half, half), :], w_for[half:])
        pb_ref[pl.ds(half, half), :] = acc_f2.astype(jnp.bfloat16)
        r_par2 = rdma(pb_ref.at[pl.ds(half, half), :],
                      po_ref.at[pl.ds(half, half), :], 5)
        r_par2.start()

        acc_my2 = block_out(xb[half:], w_mine[half:])
        acc_my = jnp.concatenate([acc_my1, acc_my2], axis=0)

        r_par1.wait()
        r_par2.wait()
        ob_ref[...] = acc_my + po_ref[...].astype(jnp.float32)
        cp_out = pltpu.make_async_copy(ob_ref, out_hbm, copy_sems.at[0])
        cp_out.start()
        cp_out.wait()

    return pl.pallas_call(
        body,
        out_shape=jax.ShapeDtypeStruct((t_per, d), jnp.float32),
        in_specs=[pl.BlockSpec(memory_space=pltpu.MemorySpace.HBM)] * 3,
        out_specs=pl.BlockSpec(memory_space=pltpu.MemorySpace.HBM),
        scratch_shapes=[
            pltpu.VMEM((e_per, d), jnp.float32),
            pltpu.VMEM((t_per, d), jnp.bfloat16),
            pltpu.VMEM((t_per, d), jnp.bfloat16),
            pltpu.VMEM((t_per, n_exp), jnp.float32),
            pltpu.VMEM((t_per, n_exp), jnp.float32),
            pltpu.VMEM((t_per, d), jnp.bfloat16),
            pltpu.VMEM((t_per, d), jnp.bfloat16),
            pltpu.VMEM((t_per + e_per, d), jnp.float32),
            pltpu.VMEM((e_per, d, f), jnp.float32),
            pltpu.VMEM((e_per, f, d), jnp.float32),
            pltpu.VMEM((d, e_per * f), jnp.bfloat16),
            pltpu.VMEM((e_per * f, d), jnp.bfloat16),
            pltpu.VMEM((t_per, d), jnp.float32),
            pltpu.SemaphoreType.DMA((6,)),
            pltpu.SemaphoreType.DMA((6,)),
            pltpu.SemaphoreType.DMA((6,)),
        ],
        compiler_params=pltpu.CompilerParams(collective_id=0),
    )(*(pltpu.with_memory_space_constraint(a, pltpu.MemorySpace.HBM)
        for a in (x_aug, W1, W2)))
